# baseline (speedup 1.0000x reference)
import functools

import jax
import jax.numpy as jnp
from jax import lax
from jax.experimental import pallas as pl
from jax.experimental.pallas import tpu as pltpu
from jax.experimental.pallas import tpu_sc as plsc

N = 10000
E = 64000
D = 128
DE = 4
NET = 5
NF = DE + 1

NC = 2
NS = 16
NW = NC * NS
EPT = E // NW
CH = 80
NCH = EPT // CH
NFH = 2
NFR = 40
FCH = 128
NPAIR = NFH * NFR * FCH
NP = 10240
RPT = NP // NS
ZR = 32
FW = 8
NPF = NP * FW
FPT = NPF // NS
Z1 = 1024
PR = NP * FW // 128
BN = 1024
BPR = BN * FW // 128

_mesh = plsc.VectorSubcoreMesh(core_axis_name="c", subcore_axis_name="s")


@functools.partial(
    pl.kernel,
    out_type=(
        jax.ShapeDtypeStruct((NC, NET, NP, D), jnp.float32),
        jax.ShapeDtypeStruct((NC * NET * NPF,), jnp.float32),
    ),
    mesh=_mesh,
    scratch_types=[
        pltpu.VMEM((2 * NCH, CH), jnp.int32),
        pltpu.VMEM((NFR, FCH), jnp.int32),
        pltpu.VMEM((NFR, FCH), jnp.float32),
        pltpu.VMEM((CH, D), jnp.float32),
        pltpu.VMEM((CH, D), jnp.float32),
        pltpu.VMEM((ZR, D), jnp.float32),
        pltpu.VMEM((Z1,), jnp.float32),
        pltpu.VMEM_SHARED((NP, D), jnp.float32),
        pltpu.VMEM_SHARED((NPF,), jnp.float32),
        pltpu.SemaphoreType.DMA,
        pltpu.SemaphoreType.DMA,
        pltpu.SemaphoreType.DMA,
        pltpu.SemaphoreType.DMA,
        pltpu.SemaphoreType.DMA,
    ],
)
def _sc_gather_scatter(op_hbm, sd_hbm, idxf_hbm, eft_hbm,
                       zg_hbm, z1_hbm,
                       outg_hbm, outf_hbm,
                       sd_v, idxf_v, eft_v, rows0_v, rows1_v, zg_v, z1_v,
                       accg, accf,
                       zsem, lsem, fsem, gsem, ssem):
    c = lax.axis_index("c")
    s = lax.axis_index("s")
    tid = c * NS + s
    rows = [rows0_v, rows1_v]

    pltpu.sync_copy(zg_hbm, zg_v)
    pltpu.sync_copy(z1_hbm, z1_v)

    for et in range(NET):
        pend = []
        for z in range(RPT // ZR):
            pend.append(pltpu.async_copy(
                zg_v, accg.at[pl.ds(s * RPT + z * ZR, ZR)], zsem))
        for z in range(FPT // Z1):
            pend.append(pltpu.async_copy(
                z1_v, accf.at[pl.ds(s * FPT + z * Z1, Z1)], zsem))
        pend.append(pltpu.async_copy(sd_hbm.at[et, tid], sd_v, lsem))
        pend.append(pltpu.async_copy(idxf_hbm.at[et, tid, 0], idxf_v, lsem))
        pend.append(pltpu.async_copy(eft_hbm.at[et, tid, 0], eft_v, lsem))
        for d in pend:
            d.wait()
        plsc.subcore_barrier()

        fds = [pltpu.async_copy(eft_v.at[r], accf.at[idxf_v.at[r]], fsem,
                                add=True)
               for r in range(NFR)]

        gd = [None] * NCH
        sdd = [None] * NCH
        gd[0] = pltpu.async_copy(op_hbm.at[sd_v.at[0]], rows[0], gsem)
        for ci in range(NCH):
            b = ci % 2
            if ci + 1 < NCH:
                if ci >= 1:
                    sdd[ci - 1].wait()
                gd[ci + 1] = pltpu.async_copy(
                    op_hbm.at[sd_v.at[ci + 1]], rows[1 - b], gsem)
            gd[ci].wait()
            sdd[ci] = pltpu.async_copy(
                rows[b], accg.at[sd_v.at[NCH + ci]], ssem, add=True)
            if ci == NCH // 2:
                for d in fds:
                    d.wait()
                l2 = [pltpu.async_copy(idxf_hbm.at[et, tid, 1], idxf_v, lsem),
                      pltpu.async_copy(eft_hbm.at[et, tid, 1], eft_v, lsem)]
                for d in l2:
                    d.wait()
                fds = [pltpu.async_copy(eft_v.at[r],
                                        accf.at[idxf_v.at[r]], fsem,
                                        add=True)
                       for r in range(NFR)]
        sdd[NCH - 2].wait()
        sdd[NCH - 1].wait()
        for d in fds:
            d.wait()
        plsc.subcore_barrier()

        pltpu.sync_copy(accg.at[pl.ds(s * RPT, RPT)],
                        outg_hbm.at[c, et, pl.ds(s * RPT, RPT)])
        pltpu.sync_copy(accf.at[pl.ds(s * FPT, FPT)],
                        outf_hbm.at[pl.ds((c * NET + et) * NPF + s * FPT, FPT)])


def _tc_body(op_ref, g_ref, f_ref, wg_ref, wfb_ref, cb_ref, o_ref):
    acc = jnp.zeros((BN, D), jnp.float32)
    for et in range(NET):
        g = g_ref[0, et] + g_ref[1, et]
        p = f_ref[0, et] + f_ref[1, et]
        sums = (jnp.dot(g, wg_ref[et], preferred_element_type=jnp.float32)
                + jnp.dot(p, wfb_ref[et],
                          preferred_element_type=jnp.float32).reshape(BN, D))
        cnt = jnp.dot(p, cb_ref[0],
                      preferred_element_type=jnp.float32).reshape(BN, D)
        acc = acc + sums / jnp.maximum(cnt, 1.0)
    o_ref[...] = jnp.maximum(op_ref[...] + acc * (1.0 / NET), 0.0)


_tc_call = pl.pallas_call(
    _tc_body,
    grid=(NP // BN,),
    in_specs=[
        pl.BlockSpec((BN, D), lambda i: (i, 0)),
        pl.BlockSpec((NC, NET, BN, D), lambda i: (0, 0, i, 0)),
        pl.BlockSpec((NC, NET, BPR, 128), lambda i: (0, 0, i, 0)),
        pl.BlockSpec((NET, D, D), lambda i: (0, 0, 0)),
        pl.BlockSpec((NET, 128, 16 * D), lambda i: (0, 0, 0)),
        pl.BlockSpec((1, 128, 16 * D), lambda i: (0, 0, 0)),
    ],
    out_specs=pl.BlockSpec((BN, D), lambda i: (i, 0)),
    out_shape=jax.ShapeDtypeStruct((N, D), jnp.float32),
)


def kernel(op_feats,
           edge_index_link, edge_feats_link, W_link, b_link,
           edge_index_prev, edge_feats_prev, W_prev, b_prev,
           edge_index_succ, edge_feats_succ, W_succ, b_succ,
           edge_index_place, edge_feats_place, W_place, b_place,
           edge_index_serve, edge_feats_serve, W_serve, b_serve):
    eis = [edge_index_link, edge_index_prev, edge_index_succ,
           edge_index_place, edge_index_serve]
    efs = [edge_feats_link, edge_feats_prev, edge_feats_succ,
           edge_feats_place, edge_feats_serve]
    Ws = [W_link, W_prev, W_succ, W_place, W_serve]
    bs = [b_link, b_prev, b_succ, b_place, b_serve]

    src = jnp.stack([ei[0] for ei in eis]).reshape(NET, NW, NCH, CH)
    dst = jnp.stack([ei[1] for ei in eis]).reshape(NET, NW, NCH, CH)
    sd = jnp.concatenate([src, dst], axis=2)
    dflat = dst.reshape(NET, NW, EPT)
    idxf = (dflat[:, :, None, :] * FW
            + jnp.arange(NF, dtype=jnp.int32)[None, None, :, None])
    idxf = idxf.reshape(NET, NW, NF * EPT)
    idxf = jnp.pad(idxf, ((0, 0), (0, 0), (0, NPAIR - NF * EPT)))
    idxf = idxf.reshape(NET, NW, NFH, NFR, FCH)
    ones = jnp.ones((1, E), jnp.float32)
    eft = jnp.stack([jnp.concatenate([f.T, ones], axis=0) for f in efs])
    eft = eft.reshape(NET, NF, NW, EPT).transpose(0, 2, 1, 3)
    eft = eft.reshape(NET, NW, NF * EPT)
    eft = jnp.pad(eft, ((0, 0), (0, 0), (0, NPAIR - NF * EPT)))
    eft = eft.reshape(NET, NW, NFH, NFR, FCH)

    zg = jnp.zeros((ZR, D), jnp.float32)
    z1 = jnp.zeros((Z1,), jnp.float32)

    outg, outf1 = _sc_gather_scatter(op_feats, sd, idxf, eft, zg, z1)
    outf = outf1.reshape(NC, NET, PR, 128)

    wg = jnp.stack([W[:D] for W in Ws])
    wf = jnp.stack([jnp.concatenate(
        [W[D:], b[None, :], jnp.zeros((FW - NF, D), jnp.float32)], axis=0)
        for W, b in zip(Ws, bs)])
    eye16 = jnp.eye(16, dtype=jnp.float32)
    wfb = jnp.stack([jnp.kron(eye16, w) for w in wf])
    csel = jnp.zeros((FW, D), jnp.float32).at[DE, :].set(1.0)
    cb = jnp.kron(eye16, csel)[None]
    return _tc_call(op_feats, outg, outf, wg, wfb, cb)


# speedup vs baseline: 1.0458x; 1.0458x over previous
import functools

import jax
import jax.numpy as jnp
from jax import lax
from jax.experimental import pallas as pl
from jax.experimental.pallas import tpu as pltpu
from jax.experimental.pallas import tpu_sc as plsc

N = 10000
E = 64000
D = 128
DE = 4
NET = 5
NF = DE + 1

NC = 2
NS = 16
NW = NC * NS
EPT = E // NW
CH = 80
NCH = EPT // CH
NFH = 2
NFR = 40
FCH = 128
NPAIR = NFH * NFR * FCH
NP = 10240
RPT = NP // NS
ZR = 32
FW = 8
NPF = NP * FW
FPT = NPF // NS
Z1 = 1024
PR = NP * FW // 128
BN = 1024
BPR = BN * FW // 128

_mesh = plsc.VectorSubcoreMesh(core_axis_name="c", subcore_axis_name="s")


@functools.partial(
    pl.kernel,
    out_type=(
        jax.ShapeDtypeStruct((NC, NET, NP, D), jnp.float32),
        jax.ShapeDtypeStruct((NC * NET * NPF,), jnp.float32),
    ),
    mesh=_mesh,
    scratch_types=[
        pltpu.VMEM((2 * NCH, CH), jnp.int32),
        pltpu.VMEM((NFR, FCH), jnp.int32),
        pltpu.VMEM((NFR, FCH), jnp.float32),
        pltpu.VMEM((CH, D), jnp.float32),
        pltpu.VMEM((CH, D), jnp.float32),
        pltpu.VMEM((ZR, D), jnp.float32),
        pltpu.VMEM((Z1,), jnp.float32),
        pltpu.VMEM_SHARED((NP, D), jnp.float32),
        pltpu.VMEM_SHARED((NPF,), jnp.float32),
        pltpu.SemaphoreType.DMA,
        pltpu.SemaphoreType.DMA,
        pltpu.SemaphoreType.DMA,
        pltpu.SemaphoreType.DMA,
        pltpu.SemaphoreType.DMA,
    ],
)
def _sc_gather_scatter(op_hbm, sd_hbm, idxf_hbm, eft_hbm,
                       zg_hbm, z1_hbm,
                       outg_hbm, outf_hbm,
                       sd_v, idxf_v, eft_v, rows0_v, rows1_v, zg_v, z1_v,
                       accg, accf,
                       zsem, lsem, fsem, gsem, ssem):
    c = lax.axis_index("c")
    s = lax.axis_index("s")
    tid = c * NS + s
    rows = [rows0_v, rows1_v]

    pltpu.sync_copy(zg_hbm, zg_v)
    pltpu.sync_copy(z1_hbm, z1_v)

    for et in range(NET):
        pend = []
        if et == 0:
            for z in range(RPT // ZR):
                pend.append(pltpu.async_copy(
                    zg_v, accg.at[pl.ds(s * RPT + z * ZR, ZR)], zsem))
            for z in range(FPT // Z1):
                pend.append(pltpu.async_copy(
                    z1_v, accf.at[pl.ds(s * FPT + z * Z1, Z1)], zsem))
        else:
            pend.append(pltpu.async_copy(
                accg.at[pl.ds(s * RPT, RPT)],
                outg_hbm.at[c, et - 1, pl.ds(s * RPT, RPT)], zsem))
            pend.append(pltpu.async_copy(
                accf.at[pl.ds(s * FPT, FPT)],
                outf_hbm.at[pl.ds(((c * NET + et - 1) * NPF + s * FPT), FPT)],
                zsem))
        pend.append(pltpu.async_copy(sd_hbm.at[et, tid], sd_v, lsem))
        pend.append(pltpu.async_copy(idxf_hbm.at[et, tid, 0], idxf_v, lsem))
        pend.append(pltpu.async_copy(eft_hbm.at[et, tid, 0], eft_v, lsem))
        for d in pend:
            d.wait()
        plsc.subcore_barrier()

        fds = [pltpu.async_copy(eft_v.at[r], accf.at[idxf_v.at[r]], fsem,
                                add=True)
               for r in range(NFR)]

        gd = [None] * NCH
        sdd = [None] * NCH
        gd[0] = pltpu.async_copy(op_hbm.at[sd_v.at[0]], rows[0], gsem)
        for ci in range(NCH):
            b = ci % 2
            if ci + 1 < NCH:
                if ci >= 1:
                    sdd[ci - 1].wait()
                gd[ci + 1] = pltpu.async_copy(
                    op_hbm.at[sd_v.at[ci + 1]], rows[1 - b], gsem)
            gd[ci].wait()
            sdd[ci] = pltpu.async_copy(
                rows[b], accg.at[sd_v.at[NCH + ci]], ssem, add=True)
            if ci == NCH // 2:
                for d in fds:
                    d.wait()
                l2 = [pltpu.async_copy(idxf_hbm.at[et, tid, 1], idxf_v, lsem),
                      pltpu.async_copy(eft_hbm.at[et, tid, 1], eft_v, lsem)]
                for d in l2:
                    d.wait()
                fds = [pltpu.async_copy(eft_v.at[r],
                                        accf.at[idxf_v.at[r]], fsem,
                                        add=True)
                       for r in range(NFR)]
        sdd[NCH - 2].wait()
        sdd[NCH - 1].wait()
        for d in fds:
            d.wait()
        plsc.subcore_barrier()

    pltpu.sync_copy(accg.at[pl.ds(s * RPT, RPT)],
                    outg_hbm.at[c, NET - 1, pl.ds(s * RPT, RPT)])
    pltpu.sync_copy(accf.at[pl.ds(s * FPT, FPT)],
                    outf_hbm.at[pl.ds(((c * NET + NET - 1) * NPF + s * FPT),
                                      FPT)])


def _tc_body(op_ref, g_ref, f_ref, wg_ref, wfb_ref, cb_ref, o_ref):
    acc = jnp.zeros((BN, D), jnp.float32)
    gprev = jnp.zeros((BN, D), jnp.float32)
    pprev = jnp.zeros((BPR, 128), jnp.float32)
    for et in range(NET):
        gpref = g_ref[0, et] + g_ref[1, et]
        ppref = f_ref[0, et] + f_ref[1, et]
        g = gpref - gprev
        p = ppref - pprev
        gprev, pprev = gpref, ppref
        sums = (jnp.dot(g, wg_ref[et], preferred_element_type=jnp.float32)
                + jnp.dot(p, wfb_ref[et],
                          preferred_element_type=jnp.float32).reshape(BN, D))
        cnt = jnp.dot(p, cb_ref[0],
                      preferred_element_type=jnp.float32).reshape(BN, D)
        acc = acc + sums / jnp.maximum(cnt, 1.0)
    o_ref[...] = jnp.maximum(op_ref[...] + acc * (1.0 / NET), 0.0)


_tc_call = pl.pallas_call(
    _tc_body,
    grid=(NP // BN,),
    in_specs=[
        pl.BlockSpec((BN, D), lambda i: (i, 0)),
        pl.BlockSpec((NC, NET, BN, D), lambda i: (0, 0, i, 0)),
        pl.BlockSpec((NC, NET, BPR, 128), lambda i: (0, 0, i, 0)),
        pl.BlockSpec((NET, D, D), lambda i: (0, 0, 0)),
        pl.BlockSpec((NET, 128, 16 * D), lambda i: (0, 0, 0)),
        pl.BlockSpec((1, 128, 16 * D), lambda i: (0, 0, 0)),
    ],
    out_specs=pl.BlockSpec((BN, D), lambda i: (i, 0)),
    out_shape=jax.ShapeDtypeStruct((N, D), jnp.float32),
)


def kernel(op_feats,
           edge_index_link, edge_feats_link, W_link, b_link,
           edge_index_prev, edge_feats_prev, W_prev, b_prev,
           edge_index_succ, edge_feats_succ, W_succ, b_succ,
           edge_index_place, edge_feats_place, W_place, b_place,
           edge_index_serve, edge_feats_serve, W_serve, b_serve):
    eis = [edge_index_link, edge_index_prev, edge_index_succ,
           edge_index_place, edge_index_serve]
    efs = [edge_feats_link, edge_feats_prev, edge_feats_succ,
           edge_feats_place, edge_feats_serve]
    Ws = [W_link, W_prev, W_succ, W_place, W_serve]
    bs = [b_link, b_prev, b_succ, b_place, b_serve]

    src = jnp.stack([ei[0] for ei in eis]).reshape(NET, NW, NCH, CH)
    dst = jnp.stack([ei[1] for ei in eis]).reshape(NET, NW, NCH, CH)
    sd = jnp.concatenate([src, dst], axis=2)
    dflat = dst.reshape(NET, NW, EPT)
    idxf = (dflat[:, :, None, :] * FW
            + jnp.arange(NF, dtype=jnp.int32)[None, None, :, None])
    idxf = idxf.reshape(NET, NW, NF * EPT)
    idxf = jnp.pad(idxf, ((0, 0), (0, 0), (0, NPAIR - NF * EPT)))
    idxf = idxf.reshape(NET, NW, NFH, NFR, FCH)
    ones = jnp.ones((1, E), jnp.float32)
    eft = jnp.stack([jnp.concatenate([f.T, ones], axis=0) for f in efs])
    eft = eft.reshape(NET, NF, NW, EPT).transpose(0, 2, 1, 3)
    eft = eft.reshape(NET, NW, NF * EPT)
    eft = jnp.pad(eft, ((0, 0), (0, 0), (0, NPAIR - NF * EPT)))
    eft = eft.reshape(NET, NW, NFH, NFR, FCH)

    zg = jnp.zeros((ZR, D), jnp.float32)
    z1 = jnp.zeros((Z1,), jnp.float32)

    outg, outf1 = _sc_gather_scatter(op_feats, sd, idxf, eft, zg, z1)
    outf = outf1.reshape(NC, NET, PR, 128)

    wg = jnp.stack([W[:D] for W in Ws])
    wf = jnp.stack([jnp.concatenate(
        [W[D:], b[None, :], jnp.zeros((FW - NF, D), jnp.float32)], axis=0)
        for W, b in zip(Ws, bs)])
    eye16 = jnp.eye(16, dtype=jnp.float32)
    wfb = jnp.stack([jnp.kron(eye16, w) for w in wf])
    csel = jnp.zeros((FW, D), jnp.float32).at[DE, :].set(1.0)
    cb = jnp.kron(eye16, csel)[None]
    return _tc_call(op_feats, outg, outf, wg, wfb, cb)
